# baseline (speedup 1.0000x reference)
import functools

import jax
import jax.numpy as jnp
from jax import lax
from jax.experimental import pallas as pl
from jax.experimental.pallas import tpu as pltpu
from jax.experimental.pallas import tpu_sc as plsc

L = 16
BLK = 256
TF = 512


def _router_body(h_ref, gw_ref, idx_ref, w_ref):
    h = h_ref[...]
    logits = lax.dot_general(h, gw_ref[...], (((1,), (1,)), ((), ())),
                             preferred_element_type=jnp.float32)
    mx = jnp.max(logits, axis=1, keepdims=True)
    ex = jnp.exp(logits - mx)
    s = ex / jnp.sum(ex, axis=1, keepdims=True)
    E = s.shape[1]
    col = lax.broadcasted_iota(jnp.int32, s.shape, 1)
    v1 = jnp.max(s, axis=1, keepdims=True)
    i1 = jnp.min(jnp.where(s == v1, col, E), axis=1, keepdims=True)
    s2 = jnp.where(col == i1, -jnp.inf, s)
    v2 = jnp.max(s2, axis=1, keepdims=True)
    i2 = jnp.min(jnp.where(s2 == v2, col, E), axis=1, keepdims=True)
    idx_ref[...] = jnp.concatenate([i1, i2], axis=1)
    w_ref[...] = jnp.concatenate([v1, v2], axis=1)


def _sort_body(ncopies, nblk, n_tok,
               fidx_hbm, fw_hbm,
               stok_hbm, wsort_hbm, pos_hbm, bexp_hbm,
               idxb, wb, histb, allhist, zb_i, zb_f, posb, pidxb, vtokb,
               bexpb, shist, sem0, sem1, sem2):
    c = lax.axis_index("c")
    sid = lax.axis_index("s")
    CH = ncopies // 16
    NV = CH // L
    m_pad = nblk * BLK
    ZCH = m_pad // 16
    lanes = lax.iota(jnp.int32, L)

    @pl.when(c == 0)
    def _work():
        base = sid * CH
        pltpu.sync_copy(fidx_hbm.at[pl.ds(base, CH)], idxb)
        pltpu.sync_copy(fw_hbm.at[pl.ds(base, CH)], wb)

        for v in range(ZCH // L):
            zb_i[pl.ds(v * L, L)] = jnp.zeros((L,), jnp.int32)
            zb_f[pl.ds(v * L, L)] = jnp.zeros((L,), jnp.float32)
        pltpu.sync_copy(zb_i, stok_hbm.at[pl.ds(sid * ZCH, ZCH)])
        pltpu.sync_copy(zb_f, wsort_hbm.at[pl.ds(sid * ZCH, ZCH)])

        zv = jnp.zeros((L,), jnp.int32)
        cnt = zv
        for v in range(NV):
            x = idxb[pl.ds(v * L, L)]
            for e in range(4):
                ce = jnp.sum((x == e).astype(jnp.int32))
                cnt = cnt + jnp.where(lanes == e, jnp.full((L,), ce, jnp.int32), zv)
        histb[...] = cnt
        pltpu.sync_copy(histb, shist.at[pl.ds(sid * L, L)])

    plsc.subcore_barrier()

    @pl.when(c == 0)
    def _work2():
        base = sid * CH
        pltpu.sync_copy(shist, allhist)
        zv = jnp.zeros((L,), jnp.int32)
        tot = zv
        pre = zv
        for t in range(16):
            row = allhist[pl.ds(t * L, L)]
            tot = tot + row
            pre = pre + jnp.where(jnp.full((L,), t < sid, jnp.bool_), row, zv)
        padded = ((tot + (BLK - 1)) // BLK) * BLK
        excl = plsc.cumsum(padded) - padded
        my_off = excl + pre

        offs = [jnp.sum(jnp.where(lanes == e, my_off, zv)) for e in range(4)]

        for v in range(NV):
            x = idxb[pl.ds(v * L, L)]
            gid = jnp.full((L,), base + v * L, jnp.int32) + lanes
            posv = zv
            for e in range(4):
                m = x == e
                mi = m.astype(jnp.int32)
                rank = plsc.cumsum(mi) - 1
                posv = jnp.where(m, jnp.full((L,), offs[e], jnp.int32) + rank, posv)
                offs[e] = offs[e] + jnp.sum(mi)
            r, o = v // 8, (v % 8) * L
            posb[r, pl.ds(o, L)] = posv
            vtokb[r, pl.ds(o, L)] = gid // 2
            pidxb[r, pl.ds(o, L)] = (gid % 2) * n_tok + gid // 2

        cps = []
        for r in range(CH // 128):
            cps.append(pltpu.async_copy(
                vtokb.at[r], stok_hbm.at[posb.at[r]], sem0))
            cps.append(pltpu.async_copy(
                wb.at[pl.ds(r * 128, 128)], wsort_hbm.at[posb.at[r]], sem1))
            cps.append(pltpu.async_copy(
                posb.at[r], pos_hbm.at[pidxb.at[r]], sem2))
        for cp in cps:
            cp.wait()

        @pl.when(sid == 0)
        def _bexp():
            segstart = excl // BLK
            ss = [jnp.sum(jnp.where(lanes == e, segstart, zv)) for e in range(4)]
            onev = jnp.full((L,), 1, jnp.int32)
            for r in range((nblk + L - 1) // L):
                bv = jnp.full((L,), r * L, jnp.int32) + lanes
                be = zv - onev
                for e in range(4):
                    be = be + jnp.where(bv >= jnp.full((L,), ss[e], jnp.int32), onev, zv)
                bexpb[pl.ds(r * L, L)] = be
            pltpu.sync_copy(bexpb.at[pl.ds(0, nblk)], bexp_hbm)


def _gather_body(m_pad, d,
                 h_hbm, stok_hbm, hs_hbm,
                 idxb, rows0, rows1, semi, sem0, sem1):
    c = lax.axis_index("c")
    sid = lax.axis_index("s")
    wid = sid * 2 + c
    RPT = m_pad // 32
    CKS = RPT // 2
    base = wid * RPT
    pltpu.async_copy(stok_hbm.at[pl.ds(base, CKS)], idxb.at[0], semi).wait()
    cp0 = pltpu.async_copy(h_hbm.at[idxb.at[0]], rows0, sem0)
    pltpu.async_copy(stok_hbm.at[pl.ds(base + CKS, CKS)], idxb.at[1], semi).wait()
    cp1 = pltpu.async_copy(h_hbm.at[idxb.at[1]], rows1, sem1)
    cp0.wait()
    pltpu.sync_copy(rows0, hs_hbm.at[pl.ds(base, CKS)])
    cp1.wait()
    pltpu.sync_copy(rows1, hs_hbm.at[pl.ds(base + CKS, CKS)])


def _gmm_body(n_shared_tiles, s_ref, hs_ref, ws_ref, wg_ref, wu_ref, wd_ref,
              h_ref, sg_ref, su_ref, sd_ref, ys_ref, sh_ref):
    b = pl.program_id(0)
    j = pl.program_id(1)

    hb = hs_ref[...].astype(jnp.bfloat16)
    g = jnp.dot(hb, wg_ref[0].astype(jnp.bfloat16), preferred_element_type=jnp.float32)
    u = jnp.dot(hb, wu_ref[0].astype(jnp.bfloat16), preferred_element_type=jnp.float32)
    t = (g * jax.nn.sigmoid(g) * u * ws_ref[...]).astype(jnp.bfloat16)
    part = jnp.dot(t, wd_ref[0].astype(jnp.bfloat16), preferred_element_type=jnp.float32)

    @pl.when(j == 0)
    def _init():
        ys_ref[...] = jnp.zeros_like(ys_ref)
    ys_ref[...] += part

    @pl.when((b == 0) & (j < n_shared_tiles))
    def _shared():
        h = h_ref[...].astype(jnp.bfloat16)
        sg = jnp.dot(h, sg_ref[...].astype(jnp.bfloat16), preferred_element_type=jnp.float32)
        su = jnp.dot(h, su_ref[...].astype(jnp.bfloat16), preferred_element_type=jnp.float32)
        st = (sg * jax.nn.sigmoid(sg) * su).astype(jnp.bfloat16)
        spart = jnp.dot(st, sd_ref[...].astype(jnp.bfloat16), preferred_element_type=jnp.float32)

        @pl.when(j == 0)
        def _init2():
            sh_ref[...] = jnp.zeros_like(sh_ref)
        sh_ref[...] += spart


def _combine_body(n_tok, d,
                  ys_hbm, pos_hbm, sh_hbm, out_hbm,
                  pa, pb, rowsa, rowsb, shb, semi, sema, semb):
    c = lax.axis_index("c")
    sid = lax.axis_index("s")
    wid = sid * 2 + c
    TPT = n_tok // 32
    CT = TPT // 2
    NVD = d // L
    base = wid * TPT
    for ch in range(2):
        cb = base + ch * CT
        pltpu.async_copy(pos_hbm.at[pl.ds(cb, CT)], pa.at[0], semi).wait()
        pltpu.async_copy(pos_hbm.at[pl.ds(n_tok + cb, CT)], pb.at[0], semi).wait()
        cpa = pltpu.async_copy(ys_hbm.at[pa.at[0]], rowsa, sema)
        cpb = pltpu.async_copy(ys_hbm.at[pb.at[0]], rowsb, semb)
        pltpu.sync_copy(sh_hbm.at[pl.ds(cb, CT)], shb)
        cpa.wait()
        cpb.wait()

        def body(t, _):
            for v in range(NVD):
                sl = pl.ds(v * L, L)
                shb[t, sl] = rowsa[t, sl] + rowsb[t, sl] + shb[t, sl]
            return 0

        lax.fori_loop(0, CT, body, 0)
        pltpu.sync_copy(shb, out_hbm.at[pl.ds(cb, CT)])


def _router_call(h, gate_weight):
    N, D = h.shape
    E = gate_weight.shape[0]
    K = 2
    return pl.pallas_call(
        _router_body,
        grid=(),
        in_specs=[pl.BlockSpec((N, D), lambda: (0, 0)),
                  pl.BlockSpec((E, D), lambda: (0, 0))],
        out_specs=[pl.BlockSpec((N, K), lambda: (0, 0)),
                   pl.BlockSpec((N, K), lambda: (0, 0))],
        out_shape=[jax.ShapeDtypeStruct((N, K), jnp.int32),
                   jax.ShapeDtypeStruct((N, K), jnp.float32)],
    )(h, gate_weight)


def _sort_call(flat_idx, flat_w):
    NC = flat_idx.shape[0]
    N = NC // 2
    K = 2
    NBLK = NC // BLK + 4
    M_PAD = NBLK * BLK
    mesh = plsc.VectorSubcoreMesh(core_axis_name="c", subcore_axis_name="s")
    CH = NC // 16
    ZCH = M_PAD // 16
    sort_fn = functools.partial(_sort_body, NC, NBLK, N)
    return pl.kernel(
        sort_fn,
        out_type=[jax.ShapeDtypeStruct((M_PAD,), jnp.int32),
                  jax.ShapeDtypeStruct((M_PAD,), jnp.float32),
                  jax.ShapeDtypeStruct((K * N,), jnp.int32),
                  jax.ShapeDtypeStruct((NBLK,), jnp.int32)],
        mesh=mesh,
        scratch_types=[
            pltpu.VMEM((CH,), jnp.int32),
            pltpu.VMEM((CH,), jnp.float32),
            pltpu.VMEM((L,), jnp.int32),
            pltpu.VMEM((16 * L,), jnp.int32),
            pltpu.VMEM((ZCH,), jnp.int32),
            pltpu.VMEM((ZCH,), jnp.float32),
            pltpu.VMEM((CH // 128, 128), jnp.int32),
            pltpu.VMEM((CH // 128, 128), jnp.int32),
            pltpu.VMEM((CH // 128, 128), jnp.int32),
            pltpu.VMEM((((NBLK + L - 1) // L) * L,), jnp.int32),
            pltpu.VMEM_SHARED((16 * L,), jnp.int32),
            pltpu.SemaphoreType.DMA,
            pltpu.SemaphoreType.DMA,
            pltpu.SemaphoreType.DMA,
        ],
        compiler_params=pltpu.CompilerParams(needs_layout_passes=False),
    )(flat_idx, flat_w)


def _gather_call(h, src_tok):
    N, D = h.shape
    M_PAD = src_tok.shape[0]
    mesh = plsc.VectorSubcoreMesh(core_axis_name="c", subcore_axis_name="s")
    gather_fn = functools.partial(_gather_body, M_PAD, D)
    CKS = M_PAD // 64
    return pl.kernel(
        gather_fn,
        out_type=jax.ShapeDtypeStruct((M_PAD, D), jnp.float32),
        mesh=mesh,
        scratch_types=[
            pltpu.VMEM((2, CKS), jnp.int32),
            pltpu.VMEM((CKS, D), jnp.float32),
            pltpu.VMEM((CKS, D), jnp.float32),
            pltpu.SemaphoreType.DMA,
            pltpu.SemaphoreType.DMA,
            pltpu.SemaphoreType.DMA,
        ],
        compiler_params=pltpu.CompilerParams(needs_layout_passes=False),
    )(h, src_tok)


def _gmm_call(blk_exp, h_sorted, ws2d, expert_gate_w, expert_up_w, expert_down_w,
              h, shared_gate_w, shared_up_w, shared_down_w):
    N, D = h.shape
    M_PAD = h_sorted.shape[0]
    NBLK = M_PAD // BLK
    FF = expert_gate_w.shape[2]
    SFF = shared_gate_w.shape[1]
    nj = FF // TF
    nsh = SFF // TF
    gmm = functools.partial(_gmm_body, nsh)
    return pl.pallas_call(
        gmm,
        grid_spec=pltpu.PrefetchScalarGridSpec(
            num_scalar_prefetch=1,
            grid=(NBLK, nj),
            in_specs=[
                pl.BlockSpec((BLK, D), lambda b, j, s: (b, 0)),
                pl.BlockSpec((BLK, 1), lambda b, j, s: (b, 0)),
                pl.BlockSpec((1, D, TF), lambda b, j, s: (s[b], 0, j)),
                pl.BlockSpec((1, D, TF), lambda b, j, s: (s[b], 0, j)),
                pl.BlockSpec((1, TF, D), lambda b, j, s: (s[b], j, 0)),
                pl.BlockSpec((N, D), lambda b, j, s: (0, 0)),
                pl.BlockSpec((D, TF), lambda b, j, s: (0, jnp.minimum(j, nsh - 1))),
                pl.BlockSpec((D, TF), lambda b, j, s: (0, jnp.minimum(j, nsh - 1))),
                pl.BlockSpec((TF, D), lambda b, j, s: (jnp.minimum(j, nsh - 1), 0)),
            ],
            out_specs=[
                pl.BlockSpec((BLK, D), lambda b, j, s: (b, 0)),
                pl.BlockSpec((N, D), lambda b, j, s: (0, 0)),
            ],
        ),
        out_shape=[jax.ShapeDtypeStruct((M_PAD, D), jnp.float32),
                   jax.ShapeDtypeStruct((N, D), jnp.float32)],
        compiler_params=pltpu.CompilerParams(
            dimension_semantics=("arbitrary", "arbitrary"),
        ),
    )(blk_exp, h_sorted, ws2d,
      expert_gate_w, expert_up_w, expert_down_w,
      h, shared_gate_w, shared_up_w, shared_down_w)


def _combine_call(y_sorted, pos, shared_out):
    N, D = shared_out.shape
    mesh = plsc.VectorSubcoreMesh(core_axis_name="c", subcore_axis_name="s")
    TPT = N // 32
    comb_fn = functools.partial(_combine_body, N, D)
    return pl.kernel(
        comb_fn,
        out_type=jax.ShapeDtypeStruct((N, D), jnp.float32),
        mesh=mesh,
        scratch_types=[
            pltpu.VMEM((1, TPT // 2), jnp.int32),
            pltpu.VMEM((1, TPT // 2), jnp.int32),
            pltpu.VMEM((TPT // 2, D), jnp.float32),
            pltpu.VMEM((TPT // 2, D), jnp.float32),
            pltpu.VMEM((TPT // 2, D), jnp.float32),
            pltpu.SemaphoreType.DMA,
            pltpu.SemaphoreType.DMA,
            pltpu.SemaphoreType.DMA,
        ],
        compiler_params=pltpu.CompilerParams(needs_layout_passes=False),
    )(y_sorted, pos, shared_out)


def kernel(hidden_states, gate_weight, expert_gate_w, expert_up_w, expert_down_w,
           shared_gate_w, shared_up_w, shared_down_w):
    B, S, D = hidden_states.shape
    N = B * S
    E = gate_weight.shape[0]
    FF = expert_gate_w.shape[2]
    SFF = shared_gate_w.shape[1]
    NC = N * 2
    NBLK = NC // BLK + E
    M_PAD = NBLK * BLK
    assert FF % TF == 0 and SFF % TF == 0 and NC % (16 * 128) == 0
    assert M_PAD % 32 == 0 and N % 32 == 0 and D % L == 0

    h = hidden_states.reshape(N, D)
    topk_idx, topk_w = _router_call(h, gate_weight)
    src_tok, w_sorted, pos, blk_exp = _sort_call(
        topk_idx.reshape(NC), topk_w.reshape(NC))
    h_sorted = _gather_call(h, src_tok)
    y_sorted, shared_out = _gmm_call(
        blk_exp, h_sorted, w_sorted.reshape(M_PAD, 1),
        expert_gate_w, expert_up_w, expert_down_w,
        h, shared_gate_w, shared_up_w, shared_down_w)
    out = _combine_call(y_sorted, pos, shared_out)
    return out.reshape(B, S, D)


# speedup vs baseline: 1.0754x; 1.0754x over previous
import functools

import jax
import jax.numpy as jnp
from jax import lax
from jax.experimental import pallas as pl
from jax.experimental.pallas import tpu as pltpu
from jax.experimental.pallas import tpu_sc as plsc

L = 16
BLK = 256
TF = 512


def _router_body(h_ref, gw_ref, idx_ref, w_ref):
    h = h_ref[...]
    logits = lax.dot_general(h, gw_ref[...], (((1,), (1,)), ((), ())),
                             preferred_element_type=jnp.float32)
    mx = jnp.max(logits, axis=1, keepdims=True)
    ex = jnp.exp(logits - mx)
    s = ex / jnp.sum(ex, axis=1, keepdims=True)
    E = s.shape[1]
    col = lax.broadcasted_iota(jnp.int32, s.shape, 1)
    v1 = jnp.max(s, axis=1, keepdims=True)
    i1 = jnp.min(jnp.where(s == v1, col, E), axis=1, keepdims=True)
    s2 = jnp.where(col == i1, -jnp.inf, s)
    v2 = jnp.max(s2, axis=1, keepdims=True)
    i2 = jnp.min(jnp.where(s2 == v2, col, E), axis=1, keepdims=True)
    idx_ref[...] = jnp.concatenate([i1, i2], axis=1)
    w_ref[...] = jnp.concatenate([v1, v2], axis=1)


def _dispatch_body(ncopies, nblk, n_tok,
                   fidx_hbm, fw_hbm, h_hbm,
                   hs_hbm, wsort_hbm, pos_hbm, bexp_hbm,
                   idxb, wb, histb, allhist, posb, pidxb, vtokb,
                   bexpb, gidx0, gidx1, rows0, rows1, shist, shtok,
                   sem0, sem1, sem2, semg0, semg1):
    c = lax.axis_index("c")
    sid = lax.axis_index("s")
    CH = ncopies // 16
    NV = CH // L
    m_pad = nblk * BLK
    lanes = lax.iota(jnp.int32, L)
    zv = jnp.zeros((L,), jnp.int32)

    base = sid * CH
    pltpu.sync_copy(fidx_hbm.at[pl.ds(base, CH)], idxb)
    pltpu.sync_copy(fw_hbm.at[pl.ds(base, CH)], wb)

    cnt = zv
    for v in range(NV):
        x = idxb[pl.ds(v * L, L)]
        for e in range(4):
            ce = jnp.sum((x == e).astype(jnp.int32))
            cnt = cnt + jnp.where(lanes == e, jnp.full((L,), ce, jnp.int32), zv)
    histb[...] = cnt
    pltpu.sync_copy(histb, shist.at[pl.ds(sid * L, L)])

    plsc.subcore_barrier()

    pltpu.sync_copy(shist, allhist)
    tot = zv
    pre = zv
    for t in range(16):
        row = allhist[pl.ds(t * L, L)]
        tot = tot + row
        pre = pre + jnp.where(jnp.full((L,), t < sid, jnp.bool_), row, zv)
    padded = ((tot + (BLK - 1)) // BLK) * BLK
    excl = plsc.cumsum(padded) - padded
    my_off = excl + pre

    offs = [jnp.sum(jnp.where(lanes == e, my_off, zv)) for e in range(4)]

    for v in range(NV):
        x = idxb[pl.ds(v * L, L)]
        gid = jnp.full((L,), base + v * L, jnp.int32) + lanes
        posv = zv
        for e in range(4):
            m = x == e
            mi = m.astype(jnp.int32)
            rank = plsc.cumsum(mi) - 1
            posv = jnp.where(m, jnp.full((L,), offs[e], jnp.int32) + rank, posv)
            offs[e] = offs[e] + jnp.sum(mi)
        r, o = v // 8, (v % 8) * L
        posb[r, pl.ds(o, L)] = posv
        vtokb[r, pl.ds(o, L)] = gid // 2
        pidxb[r, pl.ds(o, L)] = (gid % 2) * n_tok + gid // 2

    cps = []
    for r in range(CH // 128):
        cps.append(pltpu.async_copy(
            vtokb.at[r], shtok.at[posb.at[r]], sem0))
        cps.append(pltpu.async_copy(
            wb.at[pl.ds(r * 128, 128)], wsort_hbm.at[posb.at[r]], sem1))
        cps.append(pltpu.async_copy(
            posb.at[r], pos_hbm.at[pidxb.at[r]], sem2))
    for cp in cps:
        cp.wait()

    @pl.when((c == 0) & (sid == 0))
    def _bexp():
        segstart = excl // BLK
        ss = [jnp.sum(jnp.where(lanes == e, segstart, zv)) for e in range(4)]
        onev = jnp.full((L,), 1, jnp.int32)
        for r in range((nblk + L - 1) // L):
            bv = jnp.full((L,), r * L, jnp.int32) + lanes
            be = zv - onev
            for e in range(4):
                be = be + jnp.where(bv >= jnp.full((L,), ss[e], jnp.int32), onev, zv)
            bexpb[pl.ds(r * L, L)] = be
        pltpu.sync_copy(bexpb.at[pl.ds(0, nblk)], bexp_hbm)

    plsc.subcore_barrier()

    wid = sid * 2 + c
    RPT = m_pad // 32
    CKS = RPT // 2
    gbase = wid * RPT
    pltpu.sync_copy(shtok.at[pl.ds(gbase, CKS)], gidx0)
    pltpu.sync_copy(shtok.at[pl.ds(gbase + CKS, CKS)], gidx1)
    nmax = jnp.full((L,), n_tok - 1, jnp.int32)
    for v in range(CKS // L):
        sl = pl.ds(v * L, L)
        gidx0[sl] = jnp.minimum(jnp.maximum(gidx0[sl], zv), nmax)
        gidx1[sl] = jnp.minimum(jnp.maximum(gidx1[sl], zv), nmax)
    cp0 = pltpu.async_copy(h_hbm.at[gidx0], rows0, semg0)
    cp1 = pltpu.async_copy(h_hbm.at[gidx1], rows1, semg1)
    cp0.wait()
    pltpu.sync_copy(rows0, hs_hbm.at[pl.ds(gbase, CKS)])
    cp1.wait()
    pltpu.sync_copy(rows1, hs_hbm.at[pl.ds(gbase + CKS, CKS)])


def _gmm_body(n_shared_tiles, s_ref, hs_ref, ws_ref, wg_ref, wu_ref, wd_ref,
              h_ref, sg_ref, su_ref, sd_ref, ys_ref, sh_ref):
    b = pl.program_id(0)
    j = pl.program_id(1)

    hb = hs_ref[...].astype(jnp.bfloat16)
    g = jnp.dot(hb, wg_ref[0].astype(jnp.bfloat16), preferred_element_type=jnp.float32)
    u = jnp.dot(hb, wu_ref[0].astype(jnp.bfloat16), preferred_element_type=jnp.float32)
    t = (g * jax.nn.sigmoid(g) * u * ws_ref[...]).astype(jnp.bfloat16)
    part = jnp.dot(t, wd_ref[0].astype(jnp.bfloat16), preferred_element_type=jnp.float32)

    @pl.when(j == 0)
    def _init():
        ys_ref[...] = jnp.zeros_like(ys_ref)
    ys_ref[...] += part

    @pl.when((b == 0) & (j < n_shared_tiles))
    def _shared():
        h = h_ref[...].astype(jnp.bfloat16)
        sg = jnp.dot(h, sg_ref[...].astype(jnp.bfloat16), preferred_element_type=jnp.float32)
        su = jnp.dot(h, su_ref[...].astype(jnp.bfloat16), preferred_element_type=jnp.float32)
        st = (sg * jax.nn.sigmoid(sg) * su).astype(jnp.bfloat16)
        spart = jnp.dot(st, sd_ref[...].astype(jnp.bfloat16), preferred_element_type=jnp.float32)

        @pl.when(j == 0)
        def _init2():
            sh_ref[...] = jnp.zeros_like(sh_ref)
        sh_ref[...] += spart


def _combine_body(n_tok, d,
                  ys_hbm, pos_hbm, sh_hbm, out_hbm,
                  pa, pb, rowsa, rowsb, shb, semi, sema, semb):
    c = lax.axis_index("c")
    sid = lax.axis_index("s")
    wid = sid * 2 + c
    TPT = n_tok // 32
    CT = TPT // 2
    NVD = d // L
    base = wid * TPT
    for ch in range(2):
        cb = base + ch * CT
        pltpu.async_copy(pos_hbm.at[pl.ds(cb, CT)], pa.at[0], semi).wait()
        pltpu.async_copy(pos_hbm.at[pl.ds(n_tok + cb, CT)], pb.at[0], semi).wait()
        cpa = pltpu.async_copy(ys_hbm.at[pa.at[0]], rowsa, sema)
        cpb = pltpu.async_copy(ys_hbm.at[pb.at[0]], rowsb, semb)
        pltpu.sync_copy(sh_hbm.at[pl.ds(cb, CT)], shb)
        cpa.wait()
        cpb.wait()

        def body(t, _):
            for v in range(NVD):
                sl = pl.ds(v * L, L)
                shb[t, sl] = rowsa[t, sl] + rowsb[t, sl] + shb[t, sl]
            return 0

        lax.fori_loop(0, CT, body, 0)
        pltpu.sync_copy(shb, out_hbm.at[pl.ds(cb, CT)])


def _router_call(h, gate_weight):
    N, D = h.shape
    E = gate_weight.shape[0]
    K = 2
    return pl.pallas_call(
        _router_body,
        grid=(),
        in_specs=[pl.BlockSpec((N, D), lambda: (0, 0)),
                  pl.BlockSpec((E, D), lambda: (0, 0))],
        out_specs=[pl.BlockSpec((N, K), lambda: (0, 0)),
                   pl.BlockSpec((N, K), lambda: (0, 0))],
        out_shape=[jax.ShapeDtypeStruct((N, K), jnp.int32),
                   jax.ShapeDtypeStruct((N, K), jnp.float32)],
    )(h, gate_weight)


def _dispatch_call(flat_idx, flat_w, h):
    NC = flat_idx.shape[0]
    N, D = h.shape
    K = 2
    NBLK = NC // BLK + 4
    M_PAD = NBLK * BLK
    mesh = plsc.VectorSubcoreMesh(core_axis_name="c", subcore_axis_name="s")
    CH = NC // 16
    CKS = M_PAD // 64
    fn = functools.partial(_dispatch_body, NC, NBLK, N)
    return pl.kernel(
        fn,
        out_type=[jax.ShapeDtypeStruct((M_PAD, D), jnp.float32),
                  jax.ShapeDtypeStruct((M_PAD,), jnp.float32),
                  jax.ShapeDtypeStruct((K * N,), jnp.int32),
                  jax.ShapeDtypeStruct((NBLK,), jnp.int32)],
        mesh=mesh,
        scratch_types=[
            pltpu.VMEM((CH,), jnp.int32),
            pltpu.VMEM((CH,), jnp.float32),
            pltpu.VMEM((L,), jnp.int32),
            pltpu.VMEM((16 * L,), jnp.int32),
            pltpu.VMEM((CH // 128, 128), jnp.int32),
            pltpu.VMEM((CH // 128, 128), jnp.int32),
            pltpu.VMEM((CH // 128, 128), jnp.int32),
            pltpu.VMEM((((NBLK + L - 1) // L) * L,), jnp.int32),
            pltpu.VMEM((CKS,), jnp.int32),
            pltpu.VMEM((CKS,), jnp.int32),
            pltpu.VMEM((CKS, D), jnp.float32),
            pltpu.VMEM((CKS, D), jnp.float32),
            pltpu.VMEM_SHARED((16 * L,), jnp.int32),
            pltpu.VMEM_SHARED((M_PAD,), jnp.int32),
            pltpu.SemaphoreType.DMA,
            pltpu.SemaphoreType.DMA,
            pltpu.SemaphoreType.DMA,
            pltpu.SemaphoreType.DMA,
            pltpu.SemaphoreType.DMA,
        ],
        compiler_params=pltpu.CompilerParams(needs_layout_passes=False),
    )(flat_idx, flat_w, h)


def _gmm_call(blk_exp, h_sorted, ws2d, expert_gate_w, expert_up_w, expert_down_w,
              h, shared_gate_w, shared_up_w, shared_down_w):
    N, D = h.shape
    M_PAD = h_sorted.shape[0]
    NBLK = M_PAD // BLK
    FF = expert_gate_w.shape[2]
    SFF = shared_gate_w.shape[1]
    nj = FF // TF
    nsh = SFF // TF
    gmm = functools.partial(_gmm_body, nsh)
    return pl.pallas_call(
        gmm,
        grid_spec=pltpu.PrefetchScalarGridSpec(
            num_scalar_prefetch=1,
            grid=(NBLK, nj),
            in_specs=[
                pl.BlockSpec((BLK, D), lambda b, j, s: (b, 0)),
                pl.BlockSpec((BLK, 1), lambda b, j, s: (b, 0)),
                pl.BlockSpec((1, D, TF), lambda b, j, s: (s[b], 0, j)),
                pl.BlockSpec((1, D, TF), lambda b, j, s: (s[b], 0, j)),
                pl.BlockSpec((1, TF, D), lambda b, j, s: (s[b], j, 0)),
                pl.BlockSpec((N, D), lambda b, j, s: (0, 0)),
                pl.BlockSpec((D, TF), lambda b, j, s: (0, jnp.minimum(j, nsh - 1))),
                pl.BlockSpec((D, TF), lambda b, j, s: (0, jnp.minimum(j, nsh - 1))),
                pl.BlockSpec((TF, D), lambda b, j, s: (jnp.minimum(j, nsh - 1), 0)),
            ],
            out_specs=[
                pl.BlockSpec((BLK, D), lambda b, j, s: (b, 0)),
                pl.BlockSpec((N, D), lambda b, j, s: (0, 0)),
            ],
        ),
        out_shape=[jax.ShapeDtypeStruct((M_PAD, D), jnp.float32),
                   jax.ShapeDtypeStruct((N, D), jnp.float32)],
        compiler_params=pltpu.CompilerParams(
            dimension_semantics=("arbitrary", "arbitrary"),
        ),
    )(blk_exp, h_sorted, ws2d,
      expert_gate_w, expert_up_w, expert_down_w,
      h, shared_gate_w, shared_up_w, shared_down_w)


def _combine_call(y_sorted, pos, shared_out):
    N, D = shared_out.shape
    mesh = plsc.VectorSubcoreMesh(core_axis_name="c", subcore_axis_name="s")
    TPT = N // 32
    comb_fn = functools.partial(_combine_body, N, D)
    return pl.kernel(
        comb_fn,
        out_type=jax.ShapeDtypeStruct((N, D), jnp.float32),
        mesh=mesh,
        scratch_types=[
            pltpu.VMEM((1, TPT // 2), jnp.int32),
            pltpu.VMEM((1, TPT // 2), jnp.int32),
            pltpu.VMEM((TPT // 2, D), jnp.float32),
            pltpu.VMEM((TPT // 2, D), jnp.float32),
            pltpu.VMEM((TPT // 2, D), jnp.float32),
            pltpu.SemaphoreType.DMA,
            pltpu.SemaphoreType.DMA,
            pltpu.SemaphoreType.DMA,
        ],
        compiler_params=pltpu.CompilerParams(needs_layout_passes=False),
    )(y_sorted, pos, shared_out)


def kernel(hidden_states, gate_weight, expert_gate_w, expert_up_w, expert_down_w,
           shared_gate_w, shared_up_w, shared_down_w):
    B, S, D = hidden_states.shape
    N = B * S
    E = gate_weight.shape[0]
    FF = expert_gate_w.shape[2]
    SFF = shared_gate_w.shape[1]
    NC = N * 2
    NBLK = NC // BLK + E
    M_PAD = NBLK * BLK
    assert FF % TF == 0 and SFF % TF == 0 and NC % (16 * 128) == 0
    assert M_PAD % 32 == 0 and N % 32 == 0 and D % L == 0

    h = hidden_states.reshape(N, D)
    topk_idx, topk_w = _router_call(h, gate_weight)
    h_sorted, w_sorted, pos, blk_exp = _dispatch_call(
        topk_idx.reshape(NC), topk_w.reshape(NC), h)
    y_sorted, shared_out = _gmm_call(
        blk_exp, h_sorted, w_sorted.reshape(M_PAD, 1),
        expert_gate_w, expert_up_w, expert_down_w,
        h, shared_gate_w, shared_up_w, shared_down_w)
    out = _combine_call(y_sorted, pos, shared_out)
    return out.reshape(B, S, D)


# speedup vs baseline: 1.3574x; 1.2622x over previous
import functools

import jax
import jax.numpy as jnp
from jax import lax
from jax.experimental import pallas as pl
from jax.experimental.pallas import tpu as pltpu
from jax.experimental.pallas import tpu_sc as plsc

L = 16
BLK = 256
TF = 512


def _router_body(h_ref, gw_ref, idx_ref, w_ref):
    h = h_ref[...]
    logits = lax.dot_general(h, gw_ref[...], (((1,), (1,)), ((), ())),
                             preferred_element_type=jnp.float32)
    mx = jnp.max(logits, axis=1, keepdims=True)
    ex = jnp.exp(logits - mx)
    s = ex / jnp.sum(ex, axis=1, keepdims=True)
    E = s.shape[1]
    col = lax.broadcasted_iota(jnp.int32, s.shape, 1)
    v1 = jnp.max(s, axis=1, keepdims=True)
    i1 = jnp.min(jnp.where(s == v1, col, E), axis=1, keepdims=True)
    s2 = jnp.where(col == i1, -jnp.inf, s)
    v2 = jnp.max(s2, axis=1, keepdims=True)
    i2 = jnp.min(jnp.where(s2 == v2, col, E), axis=1, keepdims=True)
    idx_ref[...] = jnp.concatenate([i1, i2], axis=1)
    w_ref[...] = jnp.concatenate([v1, v2], axis=1)


def _dispatch_body(ncopies, nblk, n_tok,
                   fidx_hbm, fw_hbm, h_hbm,
                   hs_hbm, wsort_hbm, pos_hbm, bexp_hbm,
                   idxb, wb, histb, allhist, posb, pidxb, vtokb,
                   bexpb, gidx0, gidx1, rows0, rows1, shist, shtok,
                   sem0, sem1, sem2, semg0, semg1):
    c = lax.axis_index("c")
    sid = lax.axis_index("s")
    CH = ncopies // 16
    NV = CH // L
    m_pad = nblk * BLK
    lanes = lax.iota(jnp.int32, L)
    zv = jnp.zeros((L,), jnp.int32)

    base = sid * CH
    pltpu.sync_copy(fidx_hbm.at[pl.ds(base, CH)], idxb)
    pltpu.sync_copy(fw_hbm.at[pl.ds(base, CH)], wb)

    cnt = zv
    for v in range(NV):
        x = idxb[pl.ds(v * L, L)]
        for e in range(4):
            ce = jnp.sum((x == e).astype(jnp.int32))
            cnt = cnt + jnp.where(lanes == e, jnp.full((L,), ce, jnp.int32), zv)
    histb[...] = cnt
    pltpu.sync_copy(histb, shist.at[pl.ds(sid * L, L)])

    plsc.subcore_barrier()

    pltpu.sync_copy(shist, allhist)
    tot = zv
    pre = zv
    for t in range(16):
        row = allhist[pl.ds(t * L, L)]
        tot = tot + row
        pre = pre + jnp.where(jnp.full((L,), t < sid, jnp.bool_), row, zv)
    padded = ((tot + (BLK - 1)) // BLK) * BLK
    excl = plsc.cumsum(padded) - padded
    my_off = excl + pre

    offs = [jnp.sum(jnp.where(lanes == e, my_off, zv)) for e in range(4)]

    for v in range(NV):
        x = idxb[pl.ds(v * L, L)]
        gid = jnp.full((L,), base + v * L, jnp.int32) + lanes
        posv = zv
        for e in range(4):
            m = x == e
            mi = m.astype(jnp.int32)
            rank = plsc.cumsum(mi) - 1
            posv = jnp.where(m, jnp.full((L,), offs[e], jnp.int32) + rank, posv)
            offs[e] = offs[e] + jnp.sum(mi)
        r, o = v // 8, (v % 8) * L
        posb[r, pl.ds(o, L)] = posv
        vtokb[r, pl.ds(o, L)] = gid // 2
        pidxb[r, pl.ds(o, L)] = (gid % 2) * n_tok + gid // 2

    cps = []
    for r in range(CH // 128):
        cps.append(pltpu.async_copy(
            vtokb.at[r], shtok.at[posb.at[r]], sem0))
        cps.append(pltpu.async_copy(
            wb.at[pl.ds(r * 128, 128)], wsort_hbm.at[posb.at[r]], sem1))
        cps.append(pltpu.async_copy(
            posb.at[r], pos_hbm.at[pidxb.at[r]], sem2))
    for cp in cps:
        cp.wait()

    @pl.when((c == 0) & (sid == 0))
    def _bexp():
        segstart = excl // BLK
        ss = [jnp.sum(jnp.where(lanes == e, segstart, zv)) for e in range(4)]
        onev = jnp.full((L,), 1, jnp.int32)
        for r in range((nblk + L - 1) // L):
            bv = jnp.full((L,), r * L, jnp.int32) + lanes
            be = zv - onev
            for e in range(4):
                be = be + jnp.where(bv >= jnp.full((L,), ss[e], jnp.int32), onev, zv)
            bexpb[pl.ds(r * L, L)] = be
        pltpu.sync_copy(bexpb.at[pl.ds(0, nblk)], bexp_hbm)

    plsc.subcore_barrier()

    wid = sid * 2 + c
    RPT = m_pad // 32
    CKS = RPT // 2
    gbase = wid * RPT
    pltpu.sync_copy(shtok.at[pl.ds(gbase, CKS)], gidx0)
    pltpu.sync_copy(shtok.at[pl.ds(gbase + CKS, CKS)], gidx1)
    nmax = jnp.full((L,), n_tok - 1, jnp.int32)
    for v in range(CKS // L):
        sl = pl.ds(v * L, L)
        gidx0[sl] = jnp.minimum(jnp.maximum(gidx0[sl], zv), nmax)
        gidx1[sl] = jnp.minimum(jnp.maximum(gidx1[sl], zv), nmax)
    cp0 = pltpu.async_copy(h_hbm.at[gidx0], rows0, semg0)
    cp1 = pltpu.async_copy(h_hbm.at[gidx1], rows1, semg1)
    cp0.wait()
    pltpu.sync_copy(rows0, hs_hbm.at[pl.ds(gbase, CKS)])
    cp1.wait()
    pltpu.sync_copy(rows1, hs_hbm.at[pl.ds(gbase + CKS, CKS)])


def _gmm_body(n_shared_tiles, s_ref, hs_ref, ws_ref, wg_ref, wu_ref, wd_ref,
              h_ref, sg_ref, su_ref, sd_ref, ys_ref, sh_ref):
    j = pl.program_id(0)
    b = pl.program_id(1)
    blk_rows = pl.ds(b * BLK, BLK)

    hb = hs_ref[...].astype(jnp.bfloat16)
    g = jnp.dot(hb, wg_ref[0].astype(jnp.bfloat16), preferred_element_type=jnp.float32)
    u = jnp.dot(hb, wu_ref[0].astype(jnp.bfloat16), preferred_element_type=jnp.float32)
    t = (g * jax.nn.sigmoid(g) * u * ws_ref[...]).astype(jnp.bfloat16)
    part = jnp.dot(t, wd_ref[0].astype(jnp.bfloat16), preferred_element_type=jnp.float32)

    @pl.when(j == 0)
    def _init():
        ys_ref[blk_rows, :] = part

    @pl.when(j > 0)
    def _acc():
        ys_ref[blk_rows, :] += part

    @pl.when((b == 0) & (j < n_shared_tiles))
    def _shared():
        h = h_ref[...].astype(jnp.bfloat16)
        sg = jnp.dot(h, sg_ref[...].astype(jnp.bfloat16), preferred_element_type=jnp.float32)
        su = jnp.dot(h, su_ref[...].astype(jnp.bfloat16), preferred_element_type=jnp.float32)
        st = (sg * jax.nn.sigmoid(sg) * su).astype(jnp.bfloat16)
        spart = jnp.dot(st, sd_ref[...].astype(jnp.bfloat16), preferred_element_type=jnp.float32)

        @pl.when(j == 0)
        def _init2():
            sh_ref[...] = spart

        @pl.when(j > 0)
        def _acc2():
            sh_ref[...] += spart


def _combine_body(n_tok, d,
                  ys_hbm, pos_hbm, sh_hbm, out_hbm,
                  pa, pb, rowsa, rowsb, shb, semi, sema, semb):
    c = lax.axis_index("c")
    sid = lax.axis_index("s")
    wid = sid * 2 + c
    TPT = n_tok // 32
    CT = TPT // 2
    NVD = d // L
    base = wid * TPT
    for ch in range(2):
        cb = base + ch * CT
        pltpu.async_copy(pos_hbm.at[pl.ds(cb, CT)], pa.at[0], semi).wait()
        pltpu.async_copy(pos_hbm.at[pl.ds(n_tok + cb, CT)], pb.at[0], semi).wait()
        cpa = pltpu.async_copy(ys_hbm.at[pa.at[0]], rowsa, sema)
        cpb = pltpu.async_copy(ys_hbm.at[pb.at[0]], rowsb, semb)
        pltpu.sync_copy(sh_hbm.at[pl.ds(cb, CT)], shb)
        cpa.wait()
        cpb.wait()

        def body(t, _):
            for v in range(NVD):
                sl = pl.ds(v * L, L)
                shb[t, sl] = rowsa[t, sl] + rowsb[t, sl] + shb[t, sl]
            return 0

        lax.fori_loop(0, CT, body, 0)
        pltpu.sync_copy(shb, out_hbm.at[pl.ds(cb, CT)])


def _router_call(h, gate_weight):
    N, D = h.shape
    E = gate_weight.shape[0]
    K = 2
    return pl.pallas_call(
        _router_body,
        grid=(),
        in_specs=[pl.BlockSpec((N, D), lambda: (0, 0)),
                  pl.BlockSpec((E, D), lambda: (0, 0))],
        out_specs=[pl.BlockSpec((N, K), lambda: (0, 0)),
                   pl.BlockSpec((N, K), lambda: (0, 0))],
        out_shape=[jax.ShapeDtypeStruct((N, K), jnp.int32),
                   jax.ShapeDtypeStruct((N, K), jnp.float32)],
    )(h, gate_weight)


def _dispatch_call(flat_idx, flat_w, h):
    NC = flat_idx.shape[0]
    N, D = h.shape
    K = 2
    NBLK = NC // BLK + 4
    M_PAD = NBLK * BLK
    mesh = plsc.VectorSubcoreMesh(core_axis_name="c", subcore_axis_name="s")
    CH = NC // 16
    CKS = M_PAD // 64
    fn = functools.partial(_dispatch_body, NC, NBLK, N)
    return pl.kernel(
        fn,
        out_type=[jax.ShapeDtypeStruct((M_PAD, D), jnp.float32),
                  jax.ShapeDtypeStruct((M_PAD,), jnp.float32),
                  jax.ShapeDtypeStruct((K * N,), jnp.int32),
                  jax.ShapeDtypeStruct((NBLK,), jnp.int32)],
        mesh=mesh,
        scratch_types=[
            pltpu.VMEM((CH,), jnp.int32),
            pltpu.VMEM((CH,), jnp.float32),
            pltpu.VMEM((L,), jnp.int32),
            pltpu.VMEM((16 * L,), jnp.int32),
            pltpu.VMEM((CH // 128, 128), jnp.int32),
            pltpu.VMEM((CH // 128, 128), jnp.int32),
            pltpu.VMEM((CH // 128, 128), jnp.int32),
            pltpu.VMEM((((NBLK + L - 1) // L) * L,), jnp.int32),
            pltpu.VMEM((CKS,), jnp.int32),
            pltpu.VMEM((CKS,), jnp.int32),
            pltpu.VMEM((CKS, D), jnp.float32),
            pltpu.VMEM((CKS, D), jnp.float32),
            pltpu.VMEM_SHARED((16 * L,), jnp.int32),
            pltpu.VMEM_SHARED((M_PAD,), jnp.int32),
            pltpu.SemaphoreType.DMA,
            pltpu.SemaphoreType.DMA,
            pltpu.SemaphoreType.DMA,
            pltpu.SemaphoreType.DMA,
            pltpu.SemaphoreType.DMA,
        ],
        compiler_params=pltpu.CompilerParams(needs_layout_passes=False),
    )(flat_idx, flat_w, h)


def _gmm_call(blk_exp, h_sorted, ws2d, expert_gate_w, expert_up_w, expert_down_w,
              h, shared_gate_w, shared_up_w, shared_down_w):
    N, D = h.shape
    M_PAD = h_sorted.shape[0]
    NBLK = M_PAD // BLK
    FF = expert_gate_w.shape[2]
    SFF = shared_gate_w.shape[1]
    nj = FF // TF
    nsh = SFF // TF
    gmm = functools.partial(_gmm_body, nsh)
    return pl.pallas_call(
        gmm,
        grid_spec=pltpu.PrefetchScalarGridSpec(
            num_scalar_prefetch=1,
            grid=(nj, NBLK),
            in_specs=[
                pl.BlockSpec((BLK, D), lambda j, b, s: (b, 0)),
                pl.BlockSpec((BLK, 1), lambda j, b, s: (b, 0)),
                pl.BlockSpec((1, D, TF), lambda j, b, s: (s[b], 0, j)),
                pl.BlockSpec((1, D, TF), lambda j, b, s: (s[b], 0, j)),
                pl.BlockSpec((1, TF, D), lambda j, b, s: (s[b], j, 0)),
                pl.BlockSpec((N, D), lambda j, b, s: (0, 0)),
                pl.BlockSpec((D, TF), lambda j, b, s: (0, jnp.minimum(j, nsh - 1))),
                pl.BlockSpec((D, TF), lambda j, b, s: (0, jnp.minimum(j, nsh - 1))),
                pl.BlockSpec((TF, D), lambda j, b, s: (jnp.minimum(j, nsh - 1), 0)),
            ],
            out_specs=[
                pl.BlockSpec((M_PAD, D), lambda j, b, s: (0, 0)),
                pl.BlockSpec((N, D), lambda j, b, s: (0, 0)),
            ],
        ),
        out_shape=[jax.ShapeDtypeStruct((M_PAD, D), jnp.float32),
                   jax.ShapeDtypeStruct((N, D), jnp.float32)],
        compiler_params=pltpu.CompilerParams(
            dimension_semantics=("arbitrary", "arbitrary"),
        ),
    )(blk_exp, h_sorted, ws2d,
      expert_gate_w, expert_up_w, expert_down_w,
      h, shared_gate_w, shared_up_w, shared_down_w)


def _combine_call(y_sorted, pos, shared_out):
    N, D = shared_out.shape
    mesh = plsc.VectorSubcoreMesh(core_axis_name="c", subcore_axis_name="s")
    TPT = N // 32
    comb_fn = functools.partial(_combine_body, N, D)
    return pl.kernel(
        comb_fn,
        out_type=jax.ShapeDtypeStruct((N, D), jnp.float32),
        mesh=mesh,
        scratch_types=[
            pltpu.VMEM((1, TPT // 2), jnp.int32),
            pltpu.VMEM((1, TPT // 2), jnp.int32),
            pltpu.VMEM((TPT // 2, D), jnp.float32),
            pltpu.VMEM((TPT // 2, D), jnp.float32),
            pltpu.VMEM((TPT // 2, D), jnp.float32),
            pltpu.SemaphoreType.DMA,
            pltpu.SemaphoreType.DMA,
            pltpu.SemaphoreType.DMA,
        ],
        compiler_params=pltpu.CompilerParams(needs_layout_passes=False),
    )(y_sorted, pos, shared_out)


def kernel(hidden_states, gate_weight, expert_gate_w, expert_up_w, expert_down_w,
           shared_gate_w, shared_up_w, shared_down_w):
    B, S, D = hidden_states.shape
    N = B * S
    E = gate_weight.shape[0]
    FF = expert_gate_w.shape[2]
    SFF = shared_gate_w.shape[1]
    NC = N * 2
    NBLK = NC // BLK + E
    M_PAD = NBLK * BLK
    assert FF % TF == 0 and SFF % TF == 0 and NC % (16 * 128) == 0
    assert M_PAD % 32 == 0 and N % 32 == 0 and D % L == 0

    h = hidden_states.reshape(N, D)
    topk_idx, topk_w = _router_call(h, gate_weight)
    h_sorted, w_sorted, pos, blk_exp = _dispatch_call(
        topk_idx.reshape(NC), topk_w.reshape(NC), h)
    y_sorted, shared_out = _gmm_call(
        blk_exp, h_sorted, w_sorted.reshape(M_PAD, 1),
        expert_gate_w, expert_up_w, expert_down_w,
        h, shared_gate_w, shared_up_w, shared_down_w)
    out = _combine_call(y_sorted, pos, shared_out)
    return out.reshape(B, S, D)


# speedup vs baseline: 1.3863x; 1.0213x over previous
import functools

import jax
import jax.numpy as jnp
from jax import lax
from jax.experimental import pallas as pl
from jax.experimental.pallas import tpu as pltpu
from jax.experimental.pallas import tpu_sc as plsc

L = 16
BLK = 256
TF = 512


def _router_body(h_ref, gw_ref, idx_ref, w_ref):
    h = h_ref[...]
    logits = lax.dot_general(h, gw_ref[...], (((1,), (1,)), ((), ())),
                             preferred_element_type=jnp.float32)
    mx = jnp.max(logits, axis=1, keepdims=True)
    ex = jnp.exp(logits - mx)
    s = ex / jnp.sum(ex, axis=1, keepdims=True)
    E = s.shape[1]
    col = lax.broadcasted_iota(jnp.int32, s.shape, 1)
    v1 = jnp.max(s, axis=1, keepdims=True)
    i1 = jnp.min(jnp.where(s == v1, col, E), axis=1, keepdims=True)
    s2 = jnp.where(col == i1, -jnp.inf, s)
    v2 = jnp.max(s2, axis=1, keepdims=True)
    i2 = jnp.min(jnp.where(s2 == v2, col, E), axis=1, keepdims=True)
    idx_ref[...] = jnp.concatenate([i1, i2], axis=1)
    w_ref[...] = jnp.concatenate([v1, v2], axis=1)


def _dispatch_body(ncopies, nblk, n_tok,
                   fidx_hbm, fw_hbm, h_hbm,
                   hs_hbm, wsort_hbm, pos_hbm, bexp_hbm,
                   idxb, wb, histb, allhist, posb, pidxb, vtokb,
                   bexpb, gidx0, gidx1, rows0, rows1, shist, shtok,
                   sem0, sem1, sem2, semg0, semg1):
    c = lax.axis_index("c")
    sid = lax.axis_index("s")
    CH = ncopies // 16
    NV = CH // L
    m_pad = nblk * BLK
    lanes = lax.iota(jnp.int32, L)
    zv = jnp.zeros((L,), jnp.int32)

    base = sid * CH
    pltpu.sync_copy(fidx_hbm.at[pl.ds(base, CH)], idxb)
    pltpu.sync_copy(fw_hbm.at[pl.ds(base, CH)], wb)

    cnt = zv
    for v in range(NV):
        x = idxb[pl.ds(v * L, L)]
        for e in range(4):
            ce = jnp.sum((x == e).astype(jnp.int32))
            cnt = cnt + jnp.where(lanes == e, jnp.full((L,), ce, jnp.int32), zv)
    histb[...] = cnt
    pltpu.sync_copy(histb, shist.at[pl.ds(sid * L, L)])

    plsc.subcore_barrier()

    pltpu.sync_copy(shist, allhist)
    tot = zv
    pre = zv
    for t in range(16):
        row = allhist[pl.ds(t * L, L)]
        tot = tot + row
        pre = pre + jnp.where(jnp.full((L,), t < sid, jnp.bool_), row, zv)
    padded = ((tot + (BLK - 1)) // BLK) * BLK
    excl = plsc.cumsum(padded) - padded
    my_off = excl + pre

    offs = [jnp.sum(jnp.where(lanes == e, my_off, zv)) for e in range(4)]

    for v in range(NV):
        x = idxb[pl.ds(v * L, L)]
        gid = jnp.full((L,), base + v * L, jnp.int32) + lanes
        posv = zv
        for e in range(4):
            m = x == e
            mi = m.astype(jnp.int32)
            rank = plsc.cumsum(mi) - 1
            posv = jnp.where(m, jnp.full((L,), offs[e], jnp.int32) + rank, posv)
            offs[e] = offs[e] + jnp.sum(mi)
        r, o = v // 8, (v % 8) * L
        posb[r, pl.ds(o, L)] = posv
        vtokb[r, pl.ds(o, L)] = gid // 2
        pidxb[r, pl.ds(o, L)] = (gid % 2) * n_tok + gid // 2

    cps = []
    for r in range(CH // 128):
        cps.append(pltpu.async_copy(
            vtokb.at[r], shtok.at[posb.at[r]], sem0))
        cps.append(pltpu.async_copy(
            wb.at[pl.ds(r * 128, 128)], wsort_hbm.at[posb.at[r]], sem1))
        cps.append(pltpu.async_copy(
            posb.at[r], pos_hbm.at[pidxb.at[r]], sem2))
    for cp in cps:
        cp.wait()

    @pl.when((c == 0) & (sid == 0))
    def _bexp():
        segstart = excl // BLK
        ss = [jnp.sum(jnp.where(lanes == e, segstart, zv)) for e in range(4)]
        onev = jnp.full((L,), 1, jnp.int32)
        for r in range((nblk + L - 1) // L):
            bv = jnp.full((L,), r * L, jnp.int32) + lanes
            be = zv - onev
            for e in range(4):
                be = be + jnp.where(bv >= jnp.full((L,), ss[e], jnp.int32), onev, zv)
            bexpb[pl.ds(r * L, L)] = be
        pltpu.sync_copy(bexpb.at[pl.ds(0, nblk)], bexp_hbm)

    plsc.subcore_barrier()

    wid = sid * 2 + c
    RPT = m_pad // 32
    CKS = RPT // 2
    gbase = wid * RPT
    pltpu.sync_copy(shtok.at[pl.ds(gbase, CKS)], gidx0)
    pltpu.sync_copy(shtok.at[pl.ds(gbase + CKS, CKS)], gidx1)
    nmax = jnp.full((L,), n_tok - 1, jnp.int32)
    for v in range(CKS // L):
        sl = pl.ds(v * L, L)
        gidx0[sl] = jnp.minimum(jnp.maximum(gidx0[sl], zv), nmax)
        gidx1[sl] = jnp.minimum(jnp.maximum(gidx1[sl], zv), nmax)
    cp0 = pltpu.async_copy(h_hbm.at[gidx0], rows0, semg0)
    cp1 = pltpu.async_copy(h_hbm.at[gidx1], rows1, semg1)
    cp0.wait()
    pltpu.sync_copy(rows0, hs_hbm.at[pl.ds(gbase, CKS)])
    cp1.wait()
    pltpu.sync_copy(rows1, hs_hbm.at[pl.ds(gbase + CKS, CKS)])


def _gmm_body(n_shared_tiles, s_ref, hs_ref, ws_ref, wg_ref, wu_ref, wd_ref,
              h_ref, sg_ref, su_ref, sd_ref, ys_ref, sh_ref):
    j = pl.program_id(0)
    b = pl.program_id(1)
    blk_rows = pl.ds(b * BLK, BLK)

    blk_rows2 = pl.ds(b * BLK, BLK)
    hb = hs_ref[blk_rows2, :]
    g = jnp.dot(hb, wg_ref[0].astype(jnp.bfloat16), preferred_element_type=jnp.float32)
    u = jnp.dot(hb, wu_ref[0].astype(jnp.bfloat16), preferred_element_type=jnp.float32)
    t = (g * jax.nn.sigmoid(g) * u * ws_ref[...]).astype(jnp.bfloat16)
    part = jnp.dot(t, wd_ref[0].astype(jnp.bfloat16), preferred_element_type=jnp.float32)

    @pl.when(j == 0)
    def _init():
        ys_ref[blk_rows, :] = part

    @pl.when(j > 0)
    def _acc():
        ys_ref[blk_rows, :] += part

    @pl.when((b == 0) & (j < n_shared_tiles))
    def _shared():
        h = h_ref[...]
        sg = jnp.dot(h, sg_ref[...].astype(jnp.bfloat16), preferred_element_type=jnp.float32)
        su = jnp.dot(h, su_ref[...].astype(jnp.bfloat16), preferred_element_type=jnp.float32)
        st = (sg * jax.nn.sigmoid(sg) * su).astype(jnp.bfloat16)
        spart = jnp.dot(st, sd_ref[...].astype(jnp.bfloat16), preferred_element_type=jnp.float32)

        @pl.when(j == 0)
        def _init2():
            sh_ref[...] = spart

        @pl.when(j > 0)
        def _acc2():
            sh_ref[...] += spart


def _combine_body(n_tok, d,
                  ys_hbm, pos_hbm, sh_hbm, out_hbm,
                  pa, pb, rowsa, rowsb, shb, semi, sema, semb):
    c = lax.axis_index("c")
    sid = lax.axis_index("s")
    wid = sid * 2 + c
    TPT = n_tok // 32
    CT = TPT // 2
    NVD = d // L
    base = wid * TPT
    for ch in range(2):
        cb = base + ch * CT
        pltpu.async_copy(pos_hbm.at[pl.ds(cb, CT)], pa.at[0], semi).wait()
        pltpu.async_copy(pos_hbm.at[pl.ds(n_tok + cb, CT)], pb.at[0], semi).wait()
        cpa = pltpu.async_copy(ys_hbm.at[pa.at[0]], rowsa, sema)
        cpb = pltpu.async_copy(ys_hbm.at[pb.at[0]], rowsb, semb)
        pltpu.sync_copy(sh_hbm.at[pl.ds(cb, CT)], shb)
        cpa.wait()
        cpb.wait()

        def body(t, _):
            for v in range(NVD):
                sl = pl.ds(v * L, L)
                shb[t, sl] = rowsa[t, sl] + rowsb[t, sl] + shb[t, sl]
            return 0

        lax.fori_loop(0, CT, body, 0)
        pltpu.sync_copy(shb, out_hbm.at[pl.ds(cb, CT)])


def _router_call(h, gate_weight):
    N, D = h.shape
    E = gate_weight.shape[0]
    K = 2
    return pl.pallas_call(
        _router_body,
        grid=(),
        in_specs=[pl.BlockSpec((N, D), lambda: (0, 0)),
                  pl.BlockSpec((E, D), lambda: (0, 0))],
        out_specs=[pl.BlockSpec((N, K), lambda: (0, 0)),
                   pl.BlockSpec((N, K), lambda: (0, 0))],
        out_shape=[jax.ShapeDtypeStruct((N, K), jnp.int32),
                   jax.ShapeDtypeStruct((N, K), jnp.float32)],
    )(h, gate_weight)


def _dispatch_call(flat_idx, flat_w, h):
    NC = flat_idx.shape[0]
    N, D = h.shape
    K = 2
    NBLK = NC // BLK + 4
    M_PAD = NBLK * BLK
    mesh = plsc.VectorSubcoreMesh(core_axis_name="c", subcore_axis_name="s")
    CH = NC // 16
    CKS = M_PAD // 64
    fn = functools.partial(_dispatch_body, NC, NBLK, N)
    return pl.kernel(
        fn,
        out_type=[jax.ShapeDtypeStruct((M_PAD, D), jnp.float32),
                  jax.ShapeDtypeStruct((M_PAD,), jnp.float32),
                  jax.ShapeDtypeStruct((K * N,), jnp.int32),
                  jax.ShapeDtypeStruct((NBLK,), jnp.int32)],
        mesh=mesh,
        scratch_types=[
            pltpu.VMEM((CH,), jnp.int32),
            pltpu.VMEM((CH,), jnp.float32),
            pltpu.VMEM((L,), jnp.int32),
            pltpu.VMEM((16 * L,), jnp.int32),
            pltpu.VMEM((CH // 128, 128), jnp.int32),
            pltpu.VMEM((CH // 128, 128), jnp.int32),
            pltpu.VMEM((CH // 128, 128), jnp.int32),
            pltpu.VMEM((((NBLK + L - 1) // L) * L,), jnp.int32),
            pltpu.VMEM((CKS,), jnp.int32),
            pltpu.VMEM((CKS,), jnp.int32),
            pltpu.VMEM((CKS, D), jnp.float32),
            pltpu.VMEM((CKS, D), jnp.float32),
            pltpu.VMEM_SHARED((16 * L,), jnp.int32),
            pltpu.VMEM_SHARED((M_PAD,), jnp.int32),
            pltpu.SemaphoreType.DMA,
            pltpu.SemaphoreType.DMA,
            pltpu.SemaphoreType.DMA,
            pltpu.SemaphoreType.DMA,
            pltpu.SemaphoreType.DMA,
        ],
        compiler_params=pltpu.CompilerParams(needs_layout_passes=False),
    )(flat_idx, flat_w, h)


def _gmm_call(blk_exp, h_sorted, ws2d, expert_gate_w, expert_up_w, expert_down_w,
              h, shared_gate_w, shared_up_w, shared_down_w):
    N, D = h.shape
    M_PAD = h_sorted.shape[0]
    NBLK = M_PAD // BLK
    FF = expert_gate_w.shape[2]
    SFF = shared_gate_w.shape[1]
    nj = FF // TF
    nsh = SFF // TF
    gmm = functools.partial(_gmm_body, nsh)
    return pl.pallas_call(
        gmm,
        grid_spec=pltpu.PrefetchScalarGridSpec(
            num_scalar_prefetch=1,
            grid=(nj, NBLK),
            in_specs=[
                pl.BlockSpec((M_PAD, D), lambda j, b, s: (0, 0)),
                pl.BlockSpec((BLK, 1), lambda j, b, s: (b, 0)),
                pl.BlockSpec((1, D, TF), lambda j, b, s: (s[b], 0, j)),
                pl.BlockSpec((1, D, TF), lambda j, b, s: (s[b], 0, j)),
                pl.BlockSpec((1, TF, D), lambda j, b, s: (s[b], j, 0)),
                pl.BlockSpec((N, D), lambda j, b, s: (0, 0)),
                pl.BlockSpec((D, TF), lambda j, b, s: (0, jnp.minimum(j, nsh - 1))),
                pl.BlockSpec((D, TF), lambda j, b, s: (0, jnp.minimum(j, nsh - 1))),
                pl.BlockSpec((TF, D), lambda j, b, s: (jnp.minimum(j, nsh - 1), 0)),
            ],
            out_specs=[
                pl.BlockSpec((M_PAD, D), lambda j, b, s: (0, 0)),
                pl.BlockSpec((N, D), lambda j, b, s: (0, 0)),
            ],
        ),
        out_shape=[jax.ShapeDtypeStruct((M_PAD, D), jnp.float32),
                   jax.ShapeDtypeStruct((N, D), jnp.float32)],
        compiler_params=pltpu.CompilerParams(
            dimension_semantics=("arbitrary", "arbitrary"),
        ),
    )(blk_exp, h_sorted.astype(jnp.bfloat16), ws2d,
      expert_gate_w, expert_up_w, expert_down_w,
      h.astype(jnp.bfloat16), shared_gate_w, shared_up_w, shared_down_w)


def _combine_call(y_sorted, pos, shared_out):
    N, D = shared_out.shape
    mesh = plsc.VectorSubcoreMesh(core_axis_name="c", subcore_axis_name="s")
    TPT = N // 32
    comb_fn = functools.partial(_combine_body, N, D)
    return pl.kernel(
        comb_fn,
        out_type=jax.ShapeDtypeStruct((N, D), jnp.float32),
        mesh=mesh,
        scratch_types=[
            pltpu.VMEM((1, TPT // 2), jnp.int32),
            pltpu.VMEM((1, TPT // 2), jnp.int32),
            pltpu.VMEM((TPT // 2, D), jnp.float32),
            pltpu.VMEM((TPT // 2, D), jnp.float32),
            pltpu.VMEM((TPT // 2, D), jnp.float32),
            pltpu.SemaphoreType.DMA,
            pltpu.SemaphoreType.DMA,
            pltpu.SemaphoreType.DMA,
        ],
        compiler_params=pltpu.CompilerParams(needs_layout_passes=False),
    )(y_sorted, pos, shared_out)


def kernel(hidden_states, gate_weight, expert_gate_w, expert_up_w, expert_down_w,
           shared_gate_w, shared_up_w, shared_down_w):
    B, S, D = hidden_states.shape
    N = B * S
    E = gate_weight.shape[0]
    FF = expert_gate_w.shape[2]
    SFF = shared_gate_w.shape[1]
    NC = N * 2
    NBLK = NC // BLK + E
    M_PAD = NBLK * BLK
    assert FF % TF == 0 and SFF % TF == 0 and NC % (16 * 128) == 0
    assert M_PAD % 32 == 0 and N % 32 == 0 and D % L == 0

    h = hidden_states.reshape(N, D)
    topk_idx, topk_w = _router_call(h, gate_weight)
    h_sorted, w_sorted, pos, blk_exp = _dispatch_call(
        topk_idx.reshape(NC), topk_w.reshape(NC), h)
    y_sorted, shared_out = _gmm_call(
        blk_exp, h_sorted, w_sorted.reshape(M_PAD, 1),
        expert_gate_w, expert_up_w, expert_down_w,
        h, shared_gate_w, shared_up_w, shared_down_w)
    out = _combine_call(y_sorted, pos, shared_out)
    return out.reshape(B, S, D)
